# baseline (speedup 1.0000x reference)
import numpy as np
import jax
import jax.numpy as jnp
from jax import lax
from jax.experimental import pallas as pl
from jax.experimental.pallas import tpu as pltpu
from jax.experimental.pallas import tpu_sc as plsc

_VOCAB = 1000000
_D = 64
_L = 200
_B = 1024
_T = _B * _L
_NC, _NS = 2, 16
_NW = _NC * _NS
_TPW = _T // _NW
_CHUNK = 400
_NCHUNK = _TPW // _CHUNK
_GW = 80
_NG = _CHUNK // _GW
_IDXROWS = _TPW // _GW


def _pos_np():
    half = _D // 2
    positions = np.arange(_L)[:, None]
    depths = np.arange(half)[None, :] / half
    angle = positions * (1.0 / 10000.0 ** depths)
    p = np.concatenate([np.sin(angle), np.cos(angle)], axis=-1).astype(np.float32)
    return np.tile(p, (_CHUNK // _L, 1))


_POS2 = _pos_np()


def _sc_body(ids_hbm, pos_hbm, table_hbm, x_hbm, mask_hbm,
             idx_v, pos_v, rows_v, mask_v, sem):
    wid = lax.axis_index("s") * _NC + lax.axis_index("c")
    row0 = wid * _IDXROWS
    tok0 = wid * _TPW
    pltpu.sync_copy(ids_hbm.at[pl.ds(row0, _IDXROWS)], idx_v)
    pltpu.sync_copy(pos_hbm, pos_v)

    def chunk_body(k, carry):
        descs = [
            pltpu.async_copy(
                table_hbm.at[idx_v.at[k * _NG + g]],
                rows_v.at[pl.ds(g * _GW, _GW)], sem)
            for g in range(_NG)
        ]
        for dsc in descs:
            dsc.wait()

        for rr in range(_NG):
            irow = k * _NG + rr

            def grp_body(g2, carry2, rr=rr, irow=irow):
                col = g2 * 16
                idvec = idx_v[irow, pl.ds(col, 16)]
                mi = jnp.where(idvec != 0, 1, 0).astype(jnp.int32)
                mask_v[irow, pl.ds(col, 16)] = mi
                mfv = mi.astype(jnp.float32)
                for lane in range(16):
                    t = rr * _GW + col + lane
                    mf = mfv[lane]
                    for c in range(_D // 16):
                        s = pl.ds(c * 16, 16)
                        rows_v[t, s] = (rows_v[t, s] + pos_v[t, s]) * mf
                return carry2

            lax.fori_loop(0, _GW // 16, grp_body, 0)

        pltpu.sync_copy(rows_v, x_hbm.at[pl.ds(tok0 + k * _CHUNK, _CHUNK)])
        return carry

    lax.fori_loop(0, _NCHUNK, chunk_body, 0)
    pltpu.sync_copy(mask_v, mask_hbm.at[pl.ds(row0, _IDXROWS)])


_sc_call = pl.kernel(
    _sc_body,
    out_type=[jax.ShapeDtypeStruct((_T, _D), jnp.float32),
              jax.ShapeDtypeStruct((_T // _GW, _GW), jnp.int32)],
    mesh=plsc.VectorSubcoreMesh(core_axis_name="c", subcore_axis_name="s",
                                num_cores=_NC, num_subcores=_NS),
    scratch_types=[
        pltpu.VMEM((_IDXROWS, _GW), jnp.int32),
        pltpu.VMEM((_CHUNK, _D), jnp.float32),
        pltpu.VMEM((_CHUNK, _D), jnp.float32),
        pltpu.VMEM((_IDXROWS, _GW), jnp.int32),
        pltpu.SemaphoreType.DMA,
    ],
    compiler_params=pltpu.CompilerParams(use_tc_tiling_on_sc=False),
)


def kernel(input_ids, table):
    ids = input_ids.astype(jnp.int32).reshape(_T // _GW, _GW)
    pos2 = jnp.asarray(_POS2)
    x, mask = _sc_call(ids, pos2, table)
    return x.reshape(_B, _L, _D), mask.reshape(_B, _L)


# speedup vs baseline: 1.0008x; 1.0008x over previous
import numpy as np
import jax
import jax.numpy as jnp
from jax import lax
from jax.experimental import pallas as pl
from jax.experimental.pallas import tpu as pltpu
from jax.experimental.pallas import tpu_sc as plsc

_VOCAB = 1000000
_D = 64
_L = 200
_B = 1024
_T = _B * _L
_NC, _NS = 2, 16
_NW = _NC * _NS
_SPW = _B // _NW
_TPW = _SPW * _L


def _pos_np():
    half = _D // 2
    positions = np.arange(_L)[:, None]
    depths = np.arange(half)[None, :] / half
    angle = positions * (1.0 / 10000.0 ** depths)
    p = np.concatenate([np.sin(angle), np.cos(angle)], axis=-1).astype(np.float32)
    return p.reshape(-1)


_POS1 = _pos_np()


def _sc_body(ids_hbm, pos_hbm, table_hbm, x_hbm, mask_hbm,
             idx_v, pair_v, mask_v, pos_v, rows_v, out_v, sem):
    wid = lax.axis_index("s") * _NC + lax.axis_index("c")
    tok0 = wid * _TPW
    pltpu.sync_copy(ids_hbm.at[pl.ds(tok0, _TPW)], idx_v)
    pltpu.sync_copy(pos_hbm, pos_v)

    def pre_body(i, carry):
        s = pl.ds(i * 16, 16)
        v = idx_v[s]
        pair_v[s] = lax.shift_right_logical(v, 1)
        mask_v[s] = jnp.where(v != 0, 1, 0).astype(jnp.int32)
        return carry

    lax.fori_loop(0, _TPW // 16, pre_body, 0)

    def seq_body(sq, carry):
        base = sq * _L
        descs = [
            pltpu.async_copy(table_hbm.at[pair_v.at[pl.ds(base, 80)]],
                             rows_v.at[pl.ds(0, 80)], sem),
            pltpu.async_copy(table_hbm.at[pair_v.at[pl.ds(base + 80, 80)]],
                             rows_v.at[pl.ds(80, 80)], sem),
            pltpu.async_copy(table_hbm.at[pair_v.at[pl.ds(base + 160, 40)]],
                             rows_v.at[pl.ds(160, 40)], sem),
        ]
        for dsc in descs:
            dsc.wait()

        def grp_body(g2, carry2):
            col = jnp.minimum(g2 * 16, _L - 16)
            idvec = idx_v[pl.ds(base + col, 16)]
            mfv = jnp.where(idvec != 0, 1.0, 0.0).astype(jnp.float32)
            hfv = (idvec & 1) * _D
            for lane in range(16):
                t = col + lane
                mf = mfv[lane]
                half = hfv[lane]
                for c in range(_D // 16):
                    so = pl.ds(half + c * 16, 16)
                    sd = pl.ds(t * _D + c * 16, 16)
                    out_v[sd] = (rows_v[t, so] + pos_v[sd]) * mf
            return carry2

        lax.fori_loop(0, (_L + 15) // 16, grp_body, 0)

        b = wid * _SPW + sq
        pltpu.sync_copy(out_v, x_hbm.at[b])
        return carry

    lax.fori_loop(0, _SPW, seq_body, 0)
    pltpu.sync_copy(mask_v, mask_hbm.at[wid])


_sc_call = pl.kernel(
    _sc_body,
    out_type=[jax.ShapeDtypeStruct((_B, _L * _D), jnp.float32),
              jax.ShapeDtypeStruct((_NW, _TPW), jnp.int32)],
    mesh=plsc.VectorSubcoreMesh(core_axis_name="c", subcore_axis_name="s",
                                num_cores=_NC, num_subcores=_NS),
    scratch_types=[
        pltpu.VMEM((_TPW,), jnp.int32),
        pltpu.VMEM((_TPW,), jnp.int32),
        pltpu.VMEM((_TPW,), jnp.int32),
        pltpu.VMEM((_L * _D,), jnp.float32),
        pltpu.VMEM((_L, 2 * _D), jnp.float32),
        pltpu.VMEM((_L * _D,), jnp.float32),
        pltpu.SemaphoreType.DMA,
    ],
    compiler_params=pltpu.CompilerParams(use_tc_tiling_on_sc=True),
)


def kernel(input_ids, table):
    ids = input_ids.astype(jnp.int32).reshape(-1)
    pos1 = jnp.asarray(_POS1)
    pair_table = table[:_VOCAB].reshape(_VOCAB // 2, 2 * _D)
    x, mask = _sc_call(ids, pos1, pair_table)
    return x.reshape(_B, _L, _D), mask.reshape(_B, _L)


# speedup vs baseline: 1.1062x; 1.1053x over previous
import numpy as np
import jax
import jax.numpy as jnp
from jax import lax
from jax.experimental import pallas as pl
from jax.experimental.pallas import tpu as pltpu
from jax.experimental.pallas import tpu_sc as plsc

_VOCAB = 1000000
_D = 64
_L = 200
_B = 1024
_T = _B * _L
_NC, _NS = 2, 16
_NW = _NC * _NS
_SPW = _B // _NW
_TPW = _SPW * _L


def _pos_np():
    half = _D // 2
    positions = np.arange(_L)[:, None]
    depths = np.arange(half)[None, :] / half
    angle = positions * (1.0 / 10000.0 ** depths)
    p = np.concatenate([np.sin(angle), np.cos(angle)], axis=-1).astype(np.float32)
    return p.reshape(-1)


_POS1 = _pos_np()


def _sc_body(ids_hbm, pos_hbm, table_hbm, x_hbm, mask_hbm,
             idx_v, pair_v, mask_v, pos_v, rows_v, rows2_v, out_v, sem):
    wid = lax.axis_index("s") * _NC + lax.axis_index("c")
    tok0 = wid * _TPW
    pltpu.sync_copy(ids_hbm.at[pl.ds(tok0, _TPW)], idx_v)
    pltpu.sync_copy(pos_hbm, pos_v)

    def pre_body(i, carry):
        s = pl.ds(i * 16, 16)
        v = idx_v[s]
        pair_v[s] = lax.shift_right_logical(v, 1)
        mask_v[s] = jnp.where(v != 0, 1, 0).astype(jnp.int32)
        return carry

    lax.fori_loop(0, _TPW // 16, pre_body, 0)

    def issue(sq, buf):
        base = sq * _L
        return [
            pltpu.async_copy(table_hbm.at[pair_v.at[pl.ds(base, 80)]],
                             buf.at[pl.ds(0, 80)], sem),
            pltpu.async_copy(table_hbm.at[pair_v.at[pl.ds(base + 80, 80)]],
                             buf.at[pl.ds(80, 80)], sem),
            pltpu.async_copy(table_hbm.at[pair_v.at[pl.ds(base + 160, 40)]],
                             buf.at[pl.ds(160, 40)], sem),
        ]

    def drain(sq, buf):
        base = sq * _L
        pltpu.make_async_copy(table_hbm.at[pair_v.at[pl.ds(base, 80)]],
                              buf.at[pl.ds(0, 80)], sem).wait()
        pltpu.make_async_copy(table_hbm.at[pair_v.at[pl.ds(base + 80, 80)]],
                              buf.at[pl.ds(80, 80)], sem).wait()
        pltpu.make_async_copy(table_hbm.at[pair_v.at[pl.ds(base + 160, 40)]],
                              buf.at[pl.ds(160, 40)], sem).wait()

    def compute_seq(sq, buf, masked):
        base = sq * _L

        def grp_body(g2, carry2):
            col = jnp.minimum(g2 * 16, _L - 16)
            idvec = idx_v[pl.ds(base + col, 16)]
            hfv = (idvec & 1) * _D
            if masked:
                mfv = jnp.where(idvec != 0, 1.0, 0.0).astype(jnp.float32)
            for lane in range(16):
                t = col + lane
                half = hfv[lane]
                for c in range(_D // 16):
                    so = pl.ds(half + c * 16, 16)
                    sd = pl.ds(t * _D + c * 16, 16)
                    r = buf[t, so] + pos_v[sd]
                    if masked:
                        r = r * mfv[lane]
                    out_v[sd] = r
            return carry2

        lax.fori_loop(0, (_L + 15) // 16, grp_body, 0)

    def process(sq, buf):
        base = sq * _L
        mn = idx_v[pl.ds(base, 16)]
        for g in range(1, 13):
            off = min(g * 16, _L - 16)
            mn = jnp.minimum(mn, idx_v[pl.ds(base + off, 16)])
        mn0 = lax.reduce_min(mn, (0,))

        @pl.when(mn0 != 0)
        def _():
            compute_seq(sq, buf, masked=False)

        @pl.when(mn0 == 0)
        def _():
            compute_seq(sq, buf, masked=True)

        pltpu.sync_copy(out_v, x_hbm.at[wid * _SPW + sq])

    issue(0, rows_v)

    def pair_body(k2, carry):
        s0 = 2 * k2
        issue(s0 + 1, rows2_v)
        drain(s0, rows_v)
        process(s0, rows_v)
        issue(jnp.minimum(s0 + 2, _SPW - 1), rows_v)
        drain(s0 + 1, rows2_v)
        process(s0 + 1, rows2_v)
        return carry

    lax.fori_loop(0, _SPW // 2, pair_body, 0)
    drain(_SPW - 1, rows_v)
    pltpu.sync_copy(mask_v, mask_hbm.at[wid])


_sc_call = pl.kernel(
    _sc_body,
    out_type=[jax.ShapeDtypeStruct((_B, _L * _D), jnp.float32),
              jax.ShapeDtypeStruct((_NW, _TPW), jnp.int32)],
    mesh=plsc.VectorSubcoreMesh(core_axis_name="c", subcore_axis_name="s",
                                num_cores=_NC, num_subcores=_NS),
    scratch_types=[
        pltpu.VMEM((_TPW,), jnp.int32),
        pltpu.VMEM((_TPW,), jnp.int32),
        pltpu.VMEM((_TPW,), jnp.int32),
        pltpu.VMEM((_L * _D,), jnp.float32),
        pltpu.VMEM((_L, 2 * _D), jnp.float32),
        pltpu.VMEM((_L, 2 * _D), jnp.float32),
        pltpu.VMEM((_L * _D,), jnp.float32),
        pltpu.SemaphoreType.DMA,
    ],
    compiler_params=pltpu.CompilerParams(use_tc_tiling_on_sc=True,
                                         needs_layout_passes=False),
)


def kernel(input_ids, table):
    ids = input_ids.astype(jnp.int32).reshape(-1)
    pos1 = jnp.asarray(_POS1)
    pair_table = table[:_VOCAB].reshape(_VOCAB // 2, 2 * _D)
    x, mask = _sc_call(ids, pos1, pair_table)
    return x.reshape(_B, _L, _D), mask.reshape(_B, _L)


# speedup vs baseline: 1.1305x; 1.0220x over previous
import numpy as np
import jax
import jax.numpy as jnp
from jax import lax
from jax.experimental import pallas as pl
from jax.experimental.pallas import tpu as pltpu
from jax.experimental.pallas import tpu_sc as plsc

_VOCAB = 1000000
_D = 64
_L = 200
_B = 1024
_T = _B * _L
_NC, _NS = 2, 16
_NW = _NC * _NS
_SPW = _B // _NW
_TPW = _SPW * _L


def _pos_np():
    half = _D // 2
    positions = np.arange(_L)[:, None]
    depths = np.arange(half)[None, :] / half
    angle = positions * (1.0 / 10000.0 ** depths)
    p = np.concatenate([np.sin(angle), np.cos(angle)], axis=-1).astype(np.float32)
    return p.reshape(-1)


_POS1 = _pos_np()


def _sc_body(ids_hbm, pos_hbm, table_hbm, x_hbm, mask_hbm,
             idx_v, pair_v, mask_v, pos_v, rows_v, rows2_v, out_v, out2_v,
             sem, sem2):
    wid = lax.axis_index("s") * _NC + lax.axis_index("c")
    tok0 = wid * _TPW
    pltpu.sync_copy(ids_hbm.at[pl.ds(tok0, _TPW)], idx_v)
    pltpu.sync_copy(pos_hbm, pos_v)

    def pre_body(i, carry):
        s = pl.ds(i * 16, 16)
        v = idx_v[s]
        pair_v[s] = lax.shift_right_logical(v, 1)
        mask_v[s] = jnp.where(v != 0, 1, 0).astype(jnp.int32)
        return carry

    lax.fori_loop(0, _TPW // 16, pre_body, 0)

    def issue(sq, buf):
        base = sq * _L
        return [
            pltpu.async_copy(table_hbm.at[pair_v.at[pl.ds(base, 80)]],
                             buf.at[pl.ds(0, 80)], sem),
            pltpu.async_copy(table_hbm.at[pair_v.at[pl.ds(base + 80, 80)]],
                             buf.at[pl.ds(80, 80)], sem),
            pltpu.async_copy(table_hbm.at[pair_v.at[pl.ds(base + 160, 40)]],
                             buf.at[pl.ds(160, 40)], sem),
        ]

    def drain(sq, buf):
        base = sq * _L
        pltpu.make_async_copy(table_hbm.at[pair_v.at[pl.ds(base, 80)]],
                              buf.at[pl.ds(0, 80)], sem).wait()
        pltpu.make_async_copy(table_hbm.at[pair_v.at[pl.ds(base + 80, 80)]],
                              buf.at[pl.ds(80, 80)], sem).wait()
        pltpu.make_async_copy(table_hbm.at[pair_v.at[pl.ds(base + 160, 40)]],
                              buf.at[pl.ds(160, 40)], sem).wait()

    def compute_seq(sq, buf, obuf, masked):
        base = sq * _L

        def grp_body(g2, carry2):
            col = jnp.minimum(g2 * 16, _L - 16)
            idvec = idx_v[pl.ds(base + col, 16)]
            hfv = (idvec & 1) * _D
            if masked:
                mfv = jnp.where(idvec != 0, 1.0, 0.0).astype(jnp.float32)
            for lane in range(16):
                t = col + lane
                half = hfv[lane]
                for c in range(_D // 16):
                    so = pl.ds(half + c * 16, 16)
                    sd = pl.ds(t * _D + c * 16, 16)
                    r = buf[t, so] + pos_v[sd]
                    if masked:
                        r = r * mfv[lane]
                    obuf[sd] = r
            return carry2

        lax.fori_loop(0, (_L + 15) // 16, grp_body, 0)

    def process(sq, buf, obuf):
        base = sq * _L
        mn = idx_v[pl.ds(base, 16)]
        for g in range(1, 13):
            off = min(g * 16, _L - 16)
            mn = jnp.minimum(mn, idx_v[pl.ds(base + off, 16)])
        mn0 = lax.reduce_min(mn, (0,))

        @pl.when(mn0 != 0)
        def _():
            compute_seq(sq, buf, obuf, masked=False)

        @pl.when(mn0 == 0)
        def _():
            compute_seq(sq, buf, obuf, masked=True)

        pltpu.async_copy(obuf, x_hbm.at[wid * _SPW + sq], sem2)

    def drain_out(sq, obuf):
        pltpu.make_async_copy(obuf, x_hbm.at[wid * _SPW + sq], sem2).wait()

    issue(0, rows_v)

    def pair_body(k2, carry):
        s0 = 2 * k2
        issue(s0 + 1, rows2_v)
        drain(s0, rows_v)

        @pl.when(k2 > 0)
        def _():
            drain_out(s0 - 2, out_v)

        process(s0, rows_v, out_v)
        issue(jnp.minimum(s0 + 2, _SPW - 1), rows_v)
        drain(s0 + 1, rows2_v)

        @pl.when(k2 > 0)
        def _():
            drain_out(s0 - 1, out2_v)

        process(s0 + 1, rows2_v, out2_v)
        return carry

    lax.fori_loop(0, _SPW // 2, pair_body, 0)
    drain(_SPW - 1, rows_v)
    drain_out(_SPW - 2, out_v)
    drain_out(_SPW - 1, out2_v)
    pltpu.sync_copy(mask_v, mask_hbm.at[wid])


_sc_call = pl.kernel(
    _sc_body,
    out_type=[jax.ShapeDtypeStruct((_B, _L * _D), jnp.float32),
              jax.ShapeDtypeStruct((_NW, _TPW), jnp.int32)],
    mesh=plsc.VectorSubcoreMesh(core_axis_name="c", subcore_axis_name="s",
                                num_cores=_NC, num_subcores=_NS),
    scratch_types=[
        pltpu.VMEM((_TPW,), jnp.int32),
        pltpu.VMEM((_TPW,), jnp.int32),
        pltpu.VMEM((_TPW,), jnp.int32),
        pltpu.VMEM((_L * _D,), jnp.float32),
        pltpu.VMEM((_L, 2 * _D), jnp.float32),
        pltpu.VMEM((_L, 2 * _D), jnp.float32),
        pltpu.VMEM((_L * _D,), jnp.float32),
        pltpu.VMEM((_L * _D,), jnp.float32),
        pltpu.SemaphoreType.DMA,
        pltpu.SemaphoreType.DMA,
    ],
    compiler_params=pltpu.CompilerParams(use_tc_tiling_on_sc=True,
                                         needs_layout_passes=False),
)


def kernel(input_ids, table):
    ids = input_ids.astype(jnp.int32).reshape(-1)
    pos1 = jnp.asarray(_POS1)
    pair_table = table[:_VOCAB].reshape(_VOCAB // 2, 2 * _D)
    x, mask = _sc_call(ids, pos1, pair_table)
    return x.reshape(_B, _L, _D), mask.reshape(_B, _L)
